# Initial kernel scaffold; baseline (speedup 1.0000x reference)
import functools

import jax
import jax.numpy as jnp
from jax import lax
from jax.experimental import pallas as pl
from jax.experimental.pallas import tpu as pltpu
from jax.experimental.pallas import tpu_sc as plsc

N_NODES = 50000
N_REL = 17
D = 64
H = 32
NE = 800000
NEP = 819200
NW = 32
EPW1 = NEP // NW
C1 = 1024
NCH1 = EPW1 // C1
EPS2 = NEP // 16
C2 = 1024
NCH2 = EPS2 // C2
ACC_ROWS = 51200
RELP_ROWS = 24

_mesh = lambda: plsc.VectorSubcoreMesh(core_axis_name="c", subcore_axis_name="s")


def _tanh(x):
    ax = jnp.abs(x)
    e = jnp.exp(ax + ax)
    t = 1.0 - 2.0 / (e + 1.0)
    return jnp.where(x < 0, -t, t)


@functools.partial(
    pl.kernel,
    out_type=jax.ShapeDtypeStruct((NEP,), jnp.float32),
    mesh=_mesh(),
    scratch_types=[
        pltpu.VMEM((8, 128), jnp.int32),
        pltpu.VMEM((C1,), jnp.int32),
        pltpu.VMEM((C1, D), jnp.float32),
        pltpu.VMEM((RELP_ROWS, D), jnp.float32),
        pltpu.VMEM((C1,), jnp.float32),
        pltpu.SemaphoreType.DMA,
    ],
)
def _pass1(cur_hbm, relp_hbm, heads_hbm, etype_hbm, scores_hbm,
           hidx, etv, hrows, relp, sbuf, sem):
    c = lax.axis_index("c")
    s = lax.axis_index("s")
    wid = s * 2 + c
    pltpu.sync_copy(relp_hbm, relp)
    base_row = wid * (EPW1 // 128)
    base_edge = wid * EPW1

    def chunk_body(ch, carry):
        row0 = base_row + ch * (C1 // 128)
        off = base_edge + ch * C1
        pltpu.sync_copy(heads_hbm.at[pl.ds(row0, C1 // 128)], hidx)
        pltpu.sync_copy(etype_hbm.at[pl.ds(off, C1)], etv)
        descs = [
            pltpu.async_copy(cur_hbm.at[hidx.at[j]],
                             hrows.at[pl.ds(j * 128, 128)], sem)
            for j in range(C1 // 128)
        ]
        for dsc in descs:
            dsc.wait()

        def group_body(g, carry2):
            e0 = g * 16
            eloc = e0 + lax.iota(jnp.int32, 16)
            et = etv[pl.ds(e0, 16)]
            acc = jnp.zeros((16,), jnp.float32)
            for d in range(D):
                dsp = jnp.full((16,), d, jnp.int32)
                hv = plsc.load_gather(hrows, [eloc, dsp])
                rv = plsc.load_gather(relp, [et, dsp])
                acc = acc + hv * _tanh(hv + rv)
            gid = off + e0 + lax.iota(jnp.int32, 16)
            sbuf[pl.ds(e0, 16)] = jnp.where(gid < NE, acc, 0.0)
            return carry2

        lax.fori_loop(0, C1 // 16, group_body, 0)
        pltpu.sync_copy(sbuf, scores_hbm.at[pl.ds(off, C1)])
        return carry

    lax.fori_loop(0, NCH1, chunk_body, 0)


@functools.partial(
    pl.kernel,
    out_type=jax.ShapeDtypeStruct((2, N_NODES, H), jnp.float32),
    mesh=_mesh(),
    scratch_types=[
        pltpu.VMEM((8, 128), jnp.int32),
        pltpu.VMEM((8, 128), jnp.int32),
        pltpu.VMEM((C2,), jnp.float32),
        pltpu.VMEM((C2, H), jnp.float32),
        pltpu.VMEM((C2, H), jnp.float32),
        pltpu.VMEM_SHARED((ACC_ROWS, H), jnp.float32),
        pltpu.SemaphoreType.DMA,
    ],
)
def _pass2(cur2_hbm, scores_hbm, tails_hbm, heads_hbm, out_hbm,
           tidx, hidx, sv, trows, wrows, acc, sem):
    c = lax.axis_index("c")
    s = lax.axis_index("s")

    zero16 = jnp.zeros((16,), jnp.float32)
    for r in range(128):
        wrows[r, pl.ds(0, 16)] = zero16
        wrows[r, pl.ds(16, 16)] = zero16
    for q in range(ACC_ROWS // 16 // 128):
        pltpu.sync_copy(wrows.at[pl.ds(0, 128)],
                        acc.at[pl.ds(s * (ACC_ROWS // 16) + q * 128, 128)])
    plsc.subcore_barrier()

    base_row = s * (EPS2 // 128)
    base_edge = s * EPS2

    def chunk_body(ch, carry):
        row0 = base_row + ch * (C2 // 128)
        off = base_edge + ch * C2
        pltpu.sync_copy(tails_hbm.at[pl.ds(row0, C2 // 128)], tidx)
        pltpu.sync_copy(heads_hbm.at[pl.ds(row0, C2 // 128)], hidx)
        pltpu.sync_copy(scores_hbm.at[pl.ds(off, C2)], sv)
        for j in range(C2 // 128):
            for k in range(8):
                v = tidx[j, pl.ds(k * 16, 16)]
                tidx[j, pl.ds(k * 16, 16)] = v + v + c
        descs = [
            pltpu.async_copy(cur2_hbm.at[tidx.at[j]],
                             trows.at[pl.ds(j * 128, 128)], sem)
            for j in range(C2 // 128)
        ]
        for dsc in descs:
            dsc.wait()

        def wgroup(g, carry2):
            e0 = g * 16
            w = sv[pl.ds(e0, 16)]
            eloc = e0 + lax.iota(jnp.int32, 16)
            for d in range(H):
                dsp = jnp.full((16,), d, jnp.int32)
                tv = plsc.load_gather(trows, [eloc, dsp])
                plsc.store_scatter(wrows, [eloc, dsp], tv * w)
            return carry2

        lax.fori_loop(0, C2 // 16, wgroup, 0)
        for j in range(C2 // 128):
            pltpu.sync_copy(wrows.at[pl.ds(j * 128, 128)],
                            acc.at[hidx.at[j]], add=True)
        return carry

    lax.fori_loop(0, NCH2, chunk_body, 0)
    plsc.subcore_barrier()

    r0 = s * (N_NODES // 16)
    for sz, o in ((1024, 0), (1024, 1024), (1024, 2048), (53, 3072)):
        pltpu.sync_copy(acc.at[pl.ds(r0 + o, sz)],
                        out_hbm.at[c, pl.ds(r0 + o, sz)])


def _relp_body(rt_ref, w_ref, b_ref, o_ref):
    o_ref[...] = (
        jnp.dot(rt_ref[...], w_ref[...].T, preferred_element_type=jnp.float32)
        + b_ref[...]
    )


def _relproj(rt_pad, w, b):
    return pl.pallas_call(
        _relp_body,
        out_shape=jax.ShapeDtypeStruct((RELP_ROWS, D), jnp.float32),
    )(rt_pad, w, b)


_RB = 1000


def _dense_body(x_ref, n0_ref, n1_ref, w1_ref, b1_ref, w2_ref, b2_ref, o_ref):
    x = x_ref[...]
    nb = jnp.concatenate([n0_ref[...], n1_ref[...]], axis=1)
    y = (
        jnp.dot(x + nb, w1_ref[...].T, preferred_element_type=jnp.float32)
        + jnp.dot(x * nb, w2_ref[...].T, preferred_element_type=jnp.float32)
        + b1_ref[...] + b2_ref[...]
    )
    y = jnp.where(y >= 0, y, 0.01 * y)
    n = jnp.sqrt(jnp.sum(y * y, axis=1, keepdims=True))
    o_ref[...] = y / jnp.maximum(n, 1e-12)


def _dense(x, n0, n1, w1, b1, w2, b2):
    grid = N_NODES // _RB
    bs = lambda shp: pl.BlockSpec(shp, lambda i: (i, 0))
    const = lambda shp: pl.BlockSpec(shp, lambda i: (0, 0))
    return pl.pallas_call(
        _dense_body,
        grid=(grid,),
        in_specs=[
            bs((_RB, D)), bs((_RB, H)), bs((_RB, H)),
            const((D, D)), const((1, D)), const((D, D)), const((1, D)),
        ],
        out_specs=bs((_RB, D)),
        out_shape=jax.ShapeDtypeStruct((N_NODES, D), jnp.float32),
    )(x, n0, n1, w1, b1, w2, b2)


def kernel(entity_table, relation_table, rp_w0, rp_b0, rp_w1, rp_b1,
           a1w0, a1b0, a2w0, a2b0, a1w1, a1b1, a2w1, a2b1,
           edge_index, edge_type):
    heads = edge_index[0].astype(jnp.int32)
    tails = edge_index[1].astype(jnp.int32)
    et = edge_type.astype(jnp.int32)
    padz = jnp.zeros((NEP - NE,), jnp.int32)
    heads_p = jnp.concatenate([heads, padz]).reshape(NEP // 128, 128)
    tails_p = jnp.concatenate([tails, padz]).reshape(NEP // 128, 128)
    et_p = jnp.concatenate([et, padz])
    rt_pad = jnp.concatenate(
        [relation_table,
         jnp.zeros((RELP_ROWS - N_REL, D), jnp.float32)], axis=0)

    rp_w = [rp_w0, rp_w1]
    rp_b = [rp_b0, rp_b1]
    a1w = [a1w0, a1w1]
    a1b = [a1b0, a1b1]
    a2w = [a2w0, a2w1]
    a2b = [a2b0, a2b1]

    cur = entity_table
    outs = [cur]
    for i in range(2):
        relp = _relproj(rt_pad, rp_w[i], rp_b[i].reshape(1, D))
        scores = _pass1(cur, relp, heads_p, et_p)
        nb = _pass2(cur.reshape(2 * N_NODES, H), scores, tails_p, heads_p)
        cur = _dense(cur, nb[0], nb[1],
                     a1w[i], a1b[i].reshape(1, D),
                     a2w[i], a2b[i].reshape(1, D))
        outs.append(cur)
    return jnp.concatenate(outs, axis=1)


# speedup vs baseline: 1.3390x; 1.3390x over previous
import functools

import jax
import jax.numpy as jnp
from jax import lax
from jax.experimental import pallas as pl
from jax.experimental.pallas import tpu as pltpu
from jax.experimental.pallas import tpu_sc as plsc

N_NODES = 50000
N_REL = 17
D = 64
H = 32
NE = 800000
NEP = 819200
NW = 32
EPW1 = NEP // NW
C1 = 1024
NCH1 = EPW1 // C1
EPS2 = NEP // 16
C2 = 1024
NCH2 = EPS2 // C2
ACC_ROWS = 51200
RELP_ROWS = 24

_mesh = lambda: plsc.VectorSubcoreMesh(core_axis_name="c", subcore_axis_name="s")


def _tanh(x):
    ax = jnp.abs(x)
    e = jnp.exp(ax + ax)
    t = 1.0 - 2.0 / (e + 1.0)
    return jnp.where(x < 0, -t, t)


@functools.partial(
    pl.kernel,
    out_type=jax.ShapeDtypeStruct((NEP,), jnp.float32),
    mesh=_mesh(),
    compiler_params=pltpu.CompilerParams(needs_layout_passes=False, use_tc_tiling_on_sc=False),
    scratch_types=[
        pltpu.VMEM((8, 128), jnp.int32),
        pltpu.VMEM((C1,), jnp.int32),
        pltpu.VMEM((C1, D), jnp.float32),
        pltpu.VMEM((RELP_ROWS, D), jnp.float32),
        pltpu.VMEM((C1,), jnp.float32),
        pltpu.SemaphoreType.DMA,
    ],
)
def _pass1(cur_hbm, relp_hbm, heads_hbm, etype_hbm, scores_hbm,
           hidx, etv, hrows, relp, sbuf, sem):
    c = lax.axis_index("c")
    s = lax.axis_index("s")
    wid = s * 2 + c
    pltpu.sync_copy(relp_hbm, relp)
    base_row = wid * (EPW1 // 128)
    base_edge = wid * EPW1

    def chunk_body(ch, carry):
        row0 = base_row + ch * (C1 // 128)
        off = base_edge + ch * C1
        pltpu.sync_copy(heads_hbm.at[pl.ds(row0, C1 // 128)], hidx)
        pltpu.sync_copy(etype_hbm.at[pl.ds(off, C1)], etv)
        descs = [
            pltpu.async_copy(cur_hbm.at[hidx.at[j]],
                             hrows.at[pl.ds(j * 128, 128)], sem)
            for j in range(C1 // 128)
        ]
        for dsc in descs:
            dsc.wait()

        def group_body(g, carry2):
            e0 = g * 16
            eloc = e0 + lax.iota(jnp.int32, 16)
            et = etv[pl.ds(e0, 16)]
            acc = jnp.zeros((16,), jnp.float32)
            for d in range(D):
                dsp = jnp.full((16,), d, jnp.int32)
                hv = plsc.load_gather(hrows, [eloc, dsp])
                rv = plsc.load_gather(relp, [et, dsp])
                acc = acc + hv * _tanh(hv + rv)
            gid = off + e0 + lax.iota(jnp.int32, 16)
            sbuf[pl.ds(e0, 16)] = jnp.where(gid < NE, acc, 0.0)
            return carry2

        lax.fori_loop(0, C1 // 16, group_body, 0)
        pltpu.sync_copy(sbuf, scores_hbm.at[pl.ds(off, C1)])
        return carry

    lax.fori_loop(0, NCH1, chunk_body, 0)


HQ = 16


@functools.partial(
    pl.kernel,
    out_type=jax.ShapeDtypeStruct((4, N_NODES, HQ), jnp.float32),
    mesh=_mesh(),
    compiler_params=pltpu.CompilerParams(needs_layout_passes=False, use_tc_tiling_on_sc=False),
    scratch_types=[
        pltpu.VMEM((8, 128), jnp.int32),
        pltpu.VMEM((8, 128), jnp.int32),
        pltpu.VMEM((C2,), jnp.float32),
        pltpu.VMEM((C2, HQ), jnp.float32),
        pltpu.VMEM((C2, HQ), jnp.float32),
        pltpu.VMEM_SHARED((ACC_ROWS, HQ), jnp.float32),
        pltpu.SemaphoreType.DMA,
    ],
)
def _pass2(cur4_hbm, scores_hbm, tails_hbm, heads_hbm, out_hbm,
           tidx, hidx, sv, trows, wrows, acc, sem):
    c = lax.axis_index("c")
    s = lax.axis_index("s")
    base_row = s * (EPS2 // 128)
    base_edge = s * EPS2
    zero16 = jnp.zeros((16,), jnp.float32)

    for qh in range(2):
        quarter = c * 2 + qh
        plsc.subcore_barrier()
        for r in range(128):
            wrows[r, pl.ds(0, 16)] = zero16
        for q in range(ACC_ROWS // 16 // 128):
            pltpu.sync_copy(wrows.at[pl.ds(0, 128)],
                            acc.at[pl.ds(s * (ACC_ROWS // 16) + q * 128, 128)])
        plsc.subcore_barrier()

        def chunk_body(ch, carry):
            row0 = base_row + ch * (C2 // 128)
            off = base_edge + ch * C2
            pltpu.sync_copy(tails_hbm.at[pl.ds(row0, C2 // 128)], tidx)
            pltpu.sync_copy(heads_hbm.at[pl.ds(row0, C2 // 128)], hidx)
            pltpu.sync_copy(scores_hbm.at[pl.ds(off, C2)], sv)
            for j in range(C2 // 128):
                for k in range(8):
                    v = tidx[j, pl.ds(k * 16, 16)]
                    tidx[j, pl.ds(k * 16, 16)] = v * 4 + quarter
            descs = [
                pltpu.async_copy(cur4_hbm.at[tidx.at[j]],
                                 trows.at[pl.ds(j * 128, 128)], sem)
                for j in range(C2 // 128)
            ]
            for dsc in descs:
                dsc.wait()

            def wgroup(g, carry2):
                e0 = g * 16
                w = sv[pl.ds(e0, 16)]
                eloc = e0 + lax.iota(jnp.int32, 16)
                for d in range(HQ):
                    dsp = jnp.full((16,), d, jnp.int32)
                    tv = plsc.load_gather(trows, [eloc, dsp])
                    plsc.store_scatter(wrows, [eloc, dsp], tv * w)
                return carry2

            lax.fori_loop(0, C2 // 16, wgroup, 0)
            for j in range(C2 // 128):
                pltpu.sync_copy(wrows.at[pl.ds(j * 128, 128)],
                                acc.at[hidx.at[j]], add=True)
            return carry

        lax.fori_loop(0, NCH2, chunk_body, 0)
        plsc.subcore_barrier()

        r0 = s * (N_NODES // 16)
        for sz, o in ((1024, 0), (1024, 1024), (1024, 2048), (53, 3072)):
            pltpu.sync_copy(acc.at[pl.ds(r0 + o, sz)],
                            out_hbm.at[quarter, pl.ds(r0 + o, sz)])


def _relp_body(rt_ref, w_ref, b_ref, o_ref):
    o_ref[...] = (
        jnp.dot(rt_ref[...], w_ref[...].T, preferred_element_type=jnp.float32)
        + b_ref[...]
    )


def _relproj(rt_pad, w, b):
    return pl.pallas_call(
        _relp_body,
        out_shape=jax.ShapeDtypeStruct((RELP_ROWS, D), jnp.float32),
    )(rt_pad, w, b)


_RB = 1000


def _dense_body(x_ref, n0_ref, n1_ref, n2_ref, n3_ref,
                w1_ref, b1_ref, w2_ref, b2_ref, o_ref):
    x = x_ref[...]
    nb = jnp.concatenate(
        [n0_ref[...], n1_ref[...], n2_ref[...], n3_ref[...]], axis=1)
    y = (
        jnp.dot(x + nb, w1_ref[...].T, preferred_element_type=jnp.float32)
        + jnp.dot(x * nb, w2_ref[...].T, preferred_element_type=jnp.float32)
        + b1_ref[...] + b2_ref[...]
    )
    y = jnp.where(y >= 0, y, 0.01 * y)
    n = jnp.sqrt(jnp.sum(y * y, axis=1, keepdims=True))
    o_ref[...] = y / jnp.maximum(n, 1e-12)


def _dense(x, nbq, w1, b1, w2, b2):
    grid = N_NODES // _RB
    bs = lambda shp: pl.BlockSpec(shp, lambda i: (i, 0))
    const = lambda shp: pl.BlockSpec(shp, lambda i: (0, 0))
    return pl.pallas_call(
        _dense_body,
        grid=(grid,),
        in_specs=[
            bs((_RB, D)),
            bs((_RB, HQ)), bs((_RB, HQ)), bs((_RB, HQ)), bs((_RB, HQ)),
            const((D, D)), const((1, D)), const((D, D)), const((1, D)),
        ],
        out_specs=bs((_RB, D)),
        out_shape=jax.ShapeDtypeStruct((N_NODES, D), jnp.float32),
    )(x, nbq[0], nbq[1], nbq[2], nbq[3], w1, b1, w2, b2)


def kernel(entity_table, relation_table, rp_w0, rp_b0, rp_w1, rp_b1,
           a1w0, a1b0, a2w0, a2b0, a1w1, a1b1, a2w1, a2b1,
           edge_index, edge_type):
    heads = edge_index[0].astype(jnp.int32)
    tails = edge_index[1].astype(jnp.int32)
    et = edge_type.astype(jnp.int32)
    padz = jnp.zeros((NEP - NE,), jnp.int32)
    heads_p = jnp.concatenate([heads, padz]).reshape(NEP // 128, 128)
    tails_p = jnp.concatenate([tails, padz]).reshape(NEP // 128, 128)
    et_p = jnp.concatenate([et, padz])
    rt_pad = jnp.concatenate(
        [relation_table,
         jnp.zeros((RELP_ROWS - N_REL, D), jnp.float32)], axis=0)

    rp_w = [rp_w0, rp_w1]
    rp_b = [rp_b0, rp_b1]
    a1w = [a1w0, a1w1]
    a1b = [a1b0, a1b1]
    a2w = [a2w0, a2w1]
    a2b = [a2b0, a2b1]

    cur = entity_table
    outs = [cur]
    for i in range(2):
        relp = _relproj(rt_pad, rp_w[i], rp_b[i].reshape(1, D))
        scores = _pass1(cur, relp, heads_p, et_p)
        nb = _pass2(cur.reshape(4 * N_NODES, HQ), scores, tails_p, heads_p)
        cur = _dense(cur, nb,
                     a1w[i], a1b[i].reshape(1, D),
                     a2w[i], a2b[i].reshape(1, D))
        outs.append(cur)
    return jnp.concatenate(outs, axis=1)


# speedup vs baseline: 1.4748x; 1.1014x over previous
import functools

import jax
import jax.numpy as jnp
from jax import lax
from jax.experimental import pallas as pl
from jax.experimental.pallas import tpu as pltpu
from jax.experimental.pallas import tpu_sc as plsc

N_NODES = 50000
N_REL = 17
D = 64
H = 32
NE = 800000
NEP = 819200
NW = 32
EPW1 = NEP // NW
C1 = 512
NCH1 = EPW1 // C1
EPS2 = NEP // 16
C2 = 1024
NCH2 = EPS2 // C2
ACC_ROWS = 51200
RELP_ROWS = 24

_mesh = lambda: plsc.VectorSubcoreMesh(core_axis_name="c", subcore_axis_name="s")


def _tanh(x):
    ax = jnp.abs(x)
    e = jnp.exp(ax + ax)
    t = 1.0 - 2.0 / (e + 1.0)
    return jnp.where(x < 0, -t, t)


@functools.partial(
    pl.kernel,
    out_type=jax.ShapeDtypeStruct((NEP,), jnp.float32),
    mesh=_mesh(),
    compiler_params=pltpu.CompilerParams(needs_layout_passes=False, use_tc_tiling_on_sc=False),
    scratch_types=[
        pltpu.VMEM((C1 // 128, 128), jnp.int32),
        pltpu.VMEM((C1 // 128, 128), jnp.int32),
        pltpu.VMEM((C1,), jnp.int32),
        pltpu.VMEM((C1,), jnp.int32),
        pltpu.VMEM((C1, D), jnp.float32),
        pltpu.VMEM((C1, D), jnp.float32),
        pltpu.VMEM((RELP_ROWS, D), jnp.float32),
        pltpu.VMEM((C1,), jnp.float32),
        pltpu.VMEM((C1,), jnp.float32),
        pltpu.SemaphoreType.DMA,
        pltpu.SemaphoreType.DMA,
    ],
)
def _pass1(cur_hbm, relp_hbm, heads_hbm, etype_hbm, scores_hbm,
           hidxA, hidxB, etvA, etvB, hrowsA, hrowsB, relp, sbufA, sbufB,
           semA, semB):
    c = lax.axis_index("c")
    s = lax.axis_index("s")
    wid = s * 2 + c
    pltpu.sync_copy(relp_hbm, relp)
    base_row = wid * (EPW1 // 128)
    base_edge = wid * EPW1
    NSUB = C1 // 128

    def prep(hidx, etv, hrows, sem, ch):
        row0 = base_row + ch * NSUB
        off = base_edge + ch * C1
        pltpu.sync_copy(heads_hbm.at[pl.ds(row0, NSUB)], hidx)
        pltpu.sync_copy(etype_hbm.at[pl.ds(off, C1)], etv)
        for j in range(NSUB):
            pltpu.async_copy(cur_hbm.at[hidx.at[j]],
                             hrows.at[pl.ds(j * 128, 128)], sem)

    def drain(hidx, hrows, sem):
        for j in range(NSUB):
            pltpu.make_async_copy(cur_hbm.at[hidx.at[j]],
                                  hrows.at[pl.ds(j * 128, 128)], sem).wait()

    def compute(etv, hrows, sbuf, ch):
        off = base_edge + ch * C1

        def group_body(g, carry2):
            e0 = g * 16
            eloc = e0 + lax.iota(jnp.int32, 16)
            et = etv[pl.ds(e0, 16)]
            acc = jnp.zeros((16,), jnp.float32)
            for d in range(D):
                dsp = jnp.full((16,), d, jnp.int32)
                hv = plsc.load_gather(hrows, [eloc, dsp])
                rv = plsc.load_gather(relp, [et, dsp])
                acc = acc + hv * _tanh(hv + rv)
            gid = off + e0 + lax.iota(jnp.int32, 16)
            sbuf[pl.ds(e0, 16)] = jnp.where(gid < NE, acc, 0.0)
            return carry2

        lax.fori_loop(0, C1 // 16, group_body, 0)
        pltpu.sync_copy(sbuf, scores_hbm.at[pl.ds(off, C1)])

    prep(hidxA, etvA, hrowsA, semA, 0)
    prep(hidxB, etvB, hrowsB, semB, 1)

    def pair_body(i, carry):
        chA = 2 * i
        chB = 2 * i + 1
        drain(hidxA, hrowsA, semA)
        compute(etvA, hrowsA, sbufA, chA)
        prep(hidxA, etvA, hrowsA, semA, jnp.minimum(chA + 2, NCH1 - 1))
        drain(hidxB, hrowsB, semB)
        compute(etvB, hrowsB, sbufB, chB)
        prep(hidxB, etvB, hrowsB, semB, jnp.minimum(chB + 2, NCH1 - 1))
        return carry

    lax.fori_loop(0, NCH1 // 2, pair_body, 0)
    drain(hidxA, hrowsA, semA)
    drain(hidxB, hrowsB, semB)


HQ = 16


@functools.partial(
    pl.kernel,
    out_type=jax.ShapeDtypeStruct((4, N_NODES, HQ), jnp.float32),
    mesh=_mesh(),
    compiler_params=pltpu.CompilerParams(needs_layout_passes=False, use_tc_tiling_on_sc=False),
    scratch_types=[
        pltpu.VMEM((C2 // 128, 128), jnp.int32),
        pltpu.VMEM((C2 // 128, 128), jnp.int32),
        pltpu.VMEM((C2 // 128, 128), jnp.int32),
        pltpu.VMEM((C2 // 128, 128), jnp.int32),
        pltpu.VMEM((C2,), jnp.float32),
        pltpu.VMEM((C2,), jnp.float32),
        pltpu.VMEM((C2, HQ), jnp.float32),
        pltpu.VMEM((C2, HQ), jnp.float32),
        pltpu.VMEM((C2, HQ), jnp.float32),
        pltpu.VMEM((C2, HQ), jnp.float32),
        pltpu.VMEM_SHARED((ACC_ROWS, HQ), jnp.float32),
        pltpu.SemaphoreType.DMA,
        pltpu.SemaphoreType.DMA,
        pltpu.SemaphoreType.DMA,
    ],
)
def _pass2(cur4_hbm, scores_hbm, tails_hbm, heads_hbm, out_hbm,
           tidxA, tidxB, hidxA, hidxB, svA, svB, trowsA, trowsB,
           wrowsA, wrowsB, acc, semA, semB, semS):
    c = lax.axis_index("c")
    s = lax.axis_index("s")
    base_row = s * (EPS2 // 128)
    base_edge = s * EPS2
    zero16 = jnp.zeros((16,), jnp.float32)
    NSUB = C2 // 128

    for qh in range(2):
        quarter = c * 2 + qh
        plsc.subcore_barrier()
        for r in range(128):
            wrowsA[r, pl.ds(0, 16)] = zero16
        for q in range(ACC_ROWS // 16 // 128):
            pltpu.sync_copy(wrowsA.at[pl.ds(0, 128)],
                            acc.at[pl.ds(s * (ACC_ROWS // 16) + q * 128, 128)])
        plsc.subcore_barrier()

        def prep(tidx, hidx, sv, trows, sem, ch):
            row0 = base_row + ch * NSUB
            off = base_edge + ch * C2
            pltpu.sync_copy(tails_hbm.at[pl.ds(row0, NSUB)], tidx)
            pltpu.sync_copy(heads_hbm.at[pl.ds(row0, NSUB)], hidx)
            pltpu.sync_copy(scores_hbm.at[pl.ds(off, C2)], sv)
            for j in range(NSUB):
                for k in range(8):
                    v = tidx[j, pl.ds(k * 16, 16)]
                    tidx[j, pl.ds(k * 16, 16)] = v * 4 + quarter
            for j in range(NSUB):
                pltpu.async_copy(cur4_hbm.at[tidx.at[j]],
                                 trows.at[pl.ds(j * 128, 128)], sem)

        def drain(tidx, trows, sem):
            for j in range(NSUB):
                pltpu.make_async_copy(cur4_hbm.at[tidx.at[j]],
                                      trows.at[pl.ds(j * 128, 128)],
                                      sem).wait()

        def compute(sv, trows, wrows):
            def wgroup(g, carry2):
                e0 = g * 16
                w = sv[pl.ds(e0, 16)]
                eloc = e0 + lax.iota(jnp.int32, 16)
                for d in range(HQ):
                    dsp = jnp.full((16,), d, jnp.int32)
                    tv = plsc.load_gather(trows, [eloc, dsp])
                    plsc.store_scatter(wrows, [eloc, dsp], tv * w)
                return carry2

            lax.fori_loop(0, C2 // 16, wgroup, 0)

        def scatter(wrows, hidx):
            for j in range(NSUB):
                pltpu.async_copy(wrows.at[pl.ds(j * 128, 128)],
                                 acc.at[hidx.at[j]], semS, add=True)
            for j in range(NSUB):
                pltpu.make_async_copy(wrows.at[pl.ds(j * 128, 128)],
                                      acc.at[hidx.at[j]], semS).wait()

        prep(tidxA, hidxA, svA, trowsA, semA, 0)
        prep(tidxB, hidxB, svB, trowsB, semB, 1)

        def pair_body(i, carry):
            chA = 2 * i
            chB = 2 * i + 1
            drain(tidxA, trowsA, semA)
            compute(svA, trowsA, wrowsA)
            scatter(wrowsA, hidxA)
            prep(tidxA, hidxA, svA, trowsA, semA,
                 jnp.minimum(chA + 2, NCH2 - 1))
            drain(tidxB, trowsB, semB)
            compute(svB, trowsB, wrowsB)
            scatter(wrowsB, hidxB)
            prep(tidxB, hidxB, svB, trowsB, semB,
                 jnp.minimum(chB + 2, NCH2 - 1))
            return carry

        lax.fori_loop(0, NCH2 // 2, pair_body, 0)
        drain(tidxA, trowsA, semA)
        drain(tidxB, trowsB, semB)
        plsc.subcore_barrier()

        r0 = s * (N_NODES // 16)
        for sz, o in ((1024, 0), (1024, 1024), (1024, 2048), (53, 3072)):
            pltpu.sync_copy(acc.at[pl.ds(r0 + o, sz)],
                            out_hbm.at[quarter, pl.ds(r0 + o, sz)])


def _relp_body(rt_ref, w_ref, b_ref, o_ref):
    o_ref[...] = (
        jnp.dot(rt_ref[...], w_ref[...].T, preferred_element_type=jnp.float32)
        + b_ref[...]
    )


def _relproj(rt_pad, w, b):
    return pl.pallas_call(
        _relp_body,
        out_shape=jax.ShapeDtypeStruct((RELP_ROWS, D), jnp.float32),
    )(rt_pad, w, b)


_RB = 1000


def _dense_body(x_ref, n0_ref, n1_ref, n2_ref, n3_ref,
                w1_ref, b1_ref, w2_ref, b2_ref, o_ref):
    x = x_ref[...]
    nb = jnp.concatenate(
        [n0_ref[...], n1_ref[...], n2_ref[...], n3_ref[...]], axis=1)
    y = (
        jnp.dot(x + nb, w1_ref[...].T, preferred_element_type=jnp.float32)
        + jnp.dot(x * nb, w2_ref[...].T, preferred_element_type=jnp.float32)
        + b1_ref[...] + b2_ref[...]
    )
    y = jnp.where(y >= 0, y, 0.01 * y)
    n = jnp.sqrt(jnp.sum(y * y, axis=1, keepdims=True))
    o_ref[...] = y / jnp.maximum(n, 1e-12)


def _dense(x, nbq, w1, b1, w2, b2):
    grid = N_NODES // _RB
    bs = lambda shp: pl.BlockSpec(shp, lambda i: (i, 0))
    const = lambda shp: pl.BlockSpec(shp, lambda i: (0, 0))
    return pl.pallas_call(
        _dense_body,
        grid=(grid,),
        in_specs=[
            bs((_RB, D)),
            bs((_RB, HQ)), bs((_RB, HQ)), bs((_RB, HQ)), bs((_RB, HQ)),
            const((D, D)), const((1, D)), const((D, D)), const((1, D)),
        ],
        out_specs=bs((_RB, D)),
        out_shape=jax.ShapeDtypeStruct((N_NODES, D), jnp.float32),
    )(x, nbq[0], nbq[1], nbq[2], nbq[3], w1, b1, w2, b2)


def kernel(entity_table, relation_table, rp_w0, rp_b0, rp_w1, rp_b1,
           a1w0, a1b0, a2w0, a2b0, a1w1, a1b1, a2w1, a2b1,
           edge_index, edge_type):
    heads = edge_index[0].astype(jnp.int32)
    tails = edge_index[1].astype(jnp.int32)
    et = edge_type.astype(jnp.int32)
    padz = jnp.zeros((NEP - NE,), jnp.int32)
    heads_p = jnp.concatenate([heads, padz]).reshape(NEP // 128, 128)
    tails_p = jnp.concatenate([tails, padz]).reshape(NEP // 128, 128)
    et_p = jnp.concatenate([et, padz])
    rt_pad = jnp.concatenate(
        [relation_table,
         jnp.zeros((RELP_ROWS - N_REL, D), jnp.float32)], axis=0)

    rp_w = [rp_w0, rp_w1]
    rp_b = [rp_b0, rp_b1]
    a1w = [a1w0, a1w1]
    a1b = [a1b0, a1b1]
    a2w = [a2w0, a2w1]
    a2b = [a2b0, a2b1]

    cur = entity_table
    outs = [cur]
    for i in range(2):
        relp = _relproj(rt_pad, rp_w[i], rp_b[i].reshape(1, D))
        scores = _pass1(cur, relp, heads_p, et_p)
        nb = _pass2(cur.reshape(4 * N_NODES, HQ), scores, tails_p, heads_p)
        cur = _dense(cur, nb,
                     a1w[i], a1b[i].reshape(1, D),
                     a2w[i], a2b[i].reshape(1, D))
        outs.append(cur)
    return jnp.concatenate(outs, axis=1)


# speedup vs baseline: 1.9731x; 1.3379x over previous
import functools

import jax
import jax.numpy as jnp
from jax import lax
from jax.experimental import pallas as pl
from jax.experimental.pallas import tpu as pltpu
from jax.experimental.pallas import tpu_sc as plsc

N_NODES = 50000
N_REL = 17
D = 64
H = 32
NE = 800000
NEP = 819200
NW = 32
EPW1 = NEP // NW
C1 = 512
NCH1 = EPW1 // C1
EPS2 = NEP // 16
C2 = 1024
NCH2 = EPS2 // C2
ACC_ROWS = 51200
RELP_ROWS = 32

_mesh = lambda: plsc.VectorSubcoreMesh(core_axis_name="c", subcore_axis_name="s")


@functools.partial(
    pl.kernel,
    out_type=jax.ShapeDtypeStruct((NEP, D), jnp.float32),
    mesh=_mesh(),
    compiler_params=pltpu.CompilerParams(needs_layout_passes=False, use_tc_tiling_on_sc=False),
    scratch_types=[
        pltpu.VMEM((C1 // 128, 128), jnp.int32),
        pltpu.VMEM((C1 // 128, 128), jnp.int32),
        pltpu.VMEM((C1, D), jnp.float32),
        pltpu.VMEM((C1, D), jnp.float32),
        pltpu.SemaphoreType.DMA,
        pltpu.SemaphoreType.DMA,
    ],
)
def _gather1(cur_hbm, heads_hbm, hrows_hbm,
             hidxA, hidxB, hrowsA, hrowsB, semA, semB):
    c = lax.axis_index("c")
    s = lax.axis_index("s")
    wid = s * 2 + c
    base_row = wid * (EPW1 // 128)
    base_edge = wid * EPW1
    NSUB = C1 // 128

    def prep(hidx, hrows, sem, ch):
        row0 = base_row + ch * NSUB
        pltpu.sync_copy(heads_hbm.at[pl.ds(row0, NSUB)], hidx)
        for j in range(NSUB):
            pltpu.async_copy(cur_hbm.at[hidx.at[j]],
                             hrows.at[pl.ds(j * 128, 128)], sem)

    def drain(hidx, hrows, sem):
        for j in range(NSUB):
            pltpu.make_async_copy(cur_hbm.at[hidx.at[j]],
                                  hrows.at[pl.ds(j * 128, 128)], sem).wait()

    prep(hidxA, hrowsA, semA, 0)
    prep(hidxB, hrowsB, semB, 1)

    def pair_body(i, carry):
        chA = 2 * i
        chB = 2 * i + 1
        drain(hidxA, hrowsA, semA)
        pltpu.sync_copy(hrowsA, hrows_hbm.at[pl.ds(base_edge + chA * C1, C1)])
        prep(hidxA, hrowsA, semA, jnp.minimum(chA + 2, NCH1 - 1))
        drain(hidxB, hrowsB, semB)
        pltpu.sync_copy(hrowsB, hrows_hbm.at[pl.ds(base_edge + chB * C1, C1)])
        prep(hidxB, hrowsB, semB, jnp.minimum(chB + 2, NCH1 - 1))
        return carry

    lax.fori_loop(0, NCH1 // 2, pair_body, 0)
    drain(hidxA, hrowsA, semA)
    drain(hidxB, hrowsB, semB)


_SB = 2048


def _score_body(h_ref, et_ref, relp_ref, o_ref):
    i = pl.program_id(0)
    h = h_ref[...]
    et = et_ref[...]
    k = lax.broadcasted_iota(jnp.int32, (1, RELP_ROWS), 1).astype(jnp.float32)
    onehot = (et == k).astype(jnp.float32)
    r = jnp.dot(onehot, relp_ref[...], preferred_element_type=jnp.float32)
    sc = jnp.sum(h * jnp.tanh(h + r), axis=1, keepdims=True)
    gid = i * _SB + lax.broadcasted_iota(jnp.int32, (_SB, 1), 0)
    o_ref[...] = jnp.where(gid < NE, sc, 0.0)


def _score(hrows, et_f, relp):
    bs = lambda shp: pl.BlockSpec(shp, lambda i: (i, 0))
    return pl.pallas_call(
        _score_body,
        grid=(NEP // _SB,),
        in_specs=[
            bs((_SB, D)), bs((_SB, 1)),
            pl.BlockSpec((RELP_ROWS, D), lambda i: (0, 0)),
        ],
        out_specs=bs((_SB, 1)),
        out_shape=jax.ShapeDtypeStruct((NEP, 1), jnp.float32),
    )(hrows, et_f, relp)


HQ = 16


@functools.partial(
    pl.kernel,
    out_type=jax.ShapeDtypeStruct((4, N_NODES, HQ), jnp.float32),
    mesh=_mesh(),
    compiler_params=pltpu.CompilerParams(needs_layout_passes=False, use_tc_tiling_on_sc=False),
    scratch_types=[
        pltpu.VMEM((C2 // 128, 128), jnp.int32),
        pltpu.VMEM((C2 // 128, 128), jnp.int32),
        pltpu.VMEM((C2 // 128, 128), jnp.int32),
        pltpu.VMEM((C2 // 128, 128), jnp.int32),
        pltpu.VMEM((C2,), jnp.float32),
        pltpu.VMEM((C2,), jnp.float32),
        pltpu.VMEM((C2, HQ), jnp.float32),
        pltpu.VMEM((C2, HQ), jnp.float32),
        pltpu.VMEM((C2, HQ), jnp.float32),
        pltpu.VMEM((C2, HQ), jnp.float32),
        pltpu.VMEM_SHARED((ACC_ROWS, HQ), jnp.float32),
        pltpu.SemaphoreType.DMA,
        pltpu.SemaphoreType.DMA,
        pltpu.SemaphoreType.DMA,
    ],
)
def _pass2(cur4_hbm, scores_hbm, tails_hbm, heads_hbm, out_hbm,
           tidxA, tidxB, hidxA, hidxB, svA, svB, trowsA, trowsB,
           wrowsA, wrowsB, acc, semA, semB, semS):
    c = lax.axis_index("c")
    s = lax.axis_index("s")
    base_row = s * (EPS2 // 128)
    base_edge = s * EPS2
    zero16 = jnp.zeros((16,), jnp.float32)
    NSUB = C2 // 128

    for qh in range(2):
        quarter = c * 2 + qh
        plsc.subcore_barrier()
        for r in range(128):
            wrowsA[r, pl.ds(0, 16)] = zero16
        for q in range(ACC_ROWS // 16 // 128):
            pltpu.sync_copy(wrowsA.at[pl.ds(0, 128)],
                            acc.at[pl.ds(s * (ACC_ROWS // 16) + q * 128, 128)])
        plsc.subcore_barrier()

        def prep(tidx, hidx, sv, trows, sem, ch):
            row0 = base_row + ch * NSUB
            off = base_edge + ch * C2
            pltpu.sync_copy(tails_hbm.at[pl.ds(row0, NSUB)], tidx)
            pltpu.sync_copy(heads_hbm.at[pl.ds(row0, NSUB)], hidx)
            pltpu.sync_copy(scores_hbm.at[pl.ds(off, C2)], sv)
            for j in range(NSUB):
                for k in range(8):
                    v = tidx[j, pl.ds(k * 16, 16)]
                    tidx[j, pl.ds(k * 16, 16)] = v * 4 + quarter
            for j in range(NSUB):
                pltpu.async_copy(cur4_hbm.at[tidx.at[j]],
                                 trows.at[pl.ds(j * 128, 128)], sem)

        def drain(tidx, trows, sem):
            for j in range(NSUB):
                pltpu.make_async_copy(cur4_hbm.at[tidx.at[j]],
                                      trows.at[pl.ds(j * 128, 128)],
                                      sem).wait()

        def compute(sv, trows, wrows):
            def wgroup(g, carry2):
                e0 = g * 16
                w = sv[pl.ds(e0, 16)]
                eloc = e0 + lax.iota(jnp.int32, 16)
                for d in range(HQ):
                    dsp = jnp.full((16,), d, jnp.int32)
                    tv = plsc.load_gather(trows, [eloc, dsp])
                    plsc.store_scatter(wrows, [eloc, dsp], tv * w)
                return carry2

            lax.fori_loop(0, C2 // 16, wgroup, 0)

        def scatter(wrows, hidx):
            for j in range(NSUB):
                pltpu.async_copy(wrows.at[pl.ds(j * 128, 128)],
                                 acc.at[hidx.at[j]], semS, add=True)
            for j in range(NSUB):
                pltpu.make_async_copy(wrows.at[pl.ds(j * 128, 128)],
                                      acc.at[hidx.at[j]], semS).wait()

        prep(tidxA, hidxA, svA, trowsA, semA, 0)
        prep(tidxB, hidxB, svB, trowsB, semB, 1)

        def pair_body(i, carry):
            chA = 2 * i
            chB = 2 * i + 1
            drain(tidxA, trowsA, semA)
            compute(svA, trowsA, wrowsA)
            scatter(wrowsA, hidxA)
            prep(tidxA, hidxA, svA, trowsA, semA,
                 jnp.minimum(chA + 2, NCH2 - 1))
            drain(tidxB, trowsB, semB)
            compute(svB, trowsB, wrowsB)
            scatter(wrowsB, hidxB)
            prep(tidxB, hidxB, svB, trowsB, semB,
                 jnp.minimum(chB + 2, NCH2 - 1))
            return carry

        lax.fori_loop(0, NCH2 // 2, pair_body, 0)
        drain(tidxA, trowsA, semA)
        drain(tidxB, trowsB, semB)
        plsc.subcore_barrier()

        r0 = s * (N_NODES // 16)
        for sz, o in ((1024, 0), (1024, 1024), (1024, 2048), (53, 3072)):
            pltpu.sync_copy(acc.at[pl.ds(r0 + o, sz)],
                            out_hbm.at[quarter, pl.ds(r0 + o, sz)])


def _relp_body(rt_ref, w_ref, b_ref, o_ref):
    o_ref[...] = (
        jnp.dot(rt_ref[...], w_ref[...].T, preferred_element_type=jnp.float32)
        + b_ref[...]
    )


def _relproj(rt_pad, w, b):
    return pl.pallas_call(
        _relp_body,
        out_shape=jax.ShapeDtypeStruct((RELP_ROWS, D), jnp.float32),
    )(rt_pad, w, b)


_RB = 1000


def _dense_body(x_ref, n0_ref, n1_ref, n2_ref, n3_ref,
                w1_ref, b1_ref, w2_ref, b2_ref, o_ref):
    x = x_ref[...]
    nb = jnp.concatenate(
        [n0_ref[...], n1_ref[...], n2_ref[...], n3_ref[...]], axis=1)
    y = (
        jnp.dot(x + nb, w1_ref[...].T, preferred_element_type=jnp.float32)
        + jnp.dot(x * nb, w2_ref[...].T, preferred_element_type=jnp.float32)
        + b1_ref[...] + b2_ref[...]
    )
    y = jnp.where(y >= 0, y, 0.01 * y)
    n = jnp.sqrt(jnp.sum(y * y, axis=1, keepdims=True))
    o_ref[...] = y / jnp.maximum(n, 1e-12)


def _dense(x, nbq, w1, b1, w2, b2):
    grid = N_NODES // _RB
    bs = lambda shp: pl.BlockSpec(shp, lambda i: (i, 0))
    const = lambda shp: pl.BlockSpec(shp, lambda i: (0, 0))
    return pl.pallas_call(
        _dense_body,
        grid=(grid,),
        in_specs=[
            bs((_RB, D)),
            bs((_RB, HQ)), bs((_RB, HQ)), bs((_RB, HQ)), bs((_RB, HQ)),
            const((D, D)), const((1, D)), const((D, D)), const((1, D)),
        ],
        out_specs=bs((_RB, D)),
        out_shape=jax.ShapeDtypeStruct((N_NODES, D), jnp.float32),
    )(x, nbq[0], nbq[1], nbq[2], nbq[3], w1, b1, w2, b2)


def kernel(entity_table, relation_table, rp_w0, rp_b0, rp_w1, rp_b1,
           a1w0, a1b0, a2w0, a2b0, a1w1, a1b1, a2w1, a2b1,
           edge_index, edge_type):
    heads = edge_index[0].astype(jnp.int32)
    tails = edge_index[1].astype(jnp.int32)
    et = edge_type.astype(jnp.int32)
    padz = jnp.zeros((NEP - NE,), jnp.int32)
    heads_p = jnp.concatenate([heads, padz]).reshape(NEP // 128, 128)
    tails_p = jnp.concatenate([tails, padz]).reshape(NEP // 128, 128)
    et_f = jnp.concatenate([et, padz]).astype(jnp.float32).reshape(NEP, 1)
    rt_pad = jnp.concatenate(
        [relation_table,
         jnp.zeros((RELP_ROWS - N_REL, D), jnp.float32)], axis=0)

    rp_w = [rp_w0, rp_w1]
    rp_b = [rp_b0, rp_b1]
    a1w = [a1w0, a1w1]
    a1b = [a1b0, a1b1]
    a2w = [a2w0, a2w1]
    a2b = [a2b0, a2b1]

    cur = entity_table
    outs = [cur]
    for i in range(2):
        relp = _relproj(rt_pad, rp_w[i], rp_b[i].reshape(1, D))
        hrows = _gather1(cur, heads_p)
        scores = _score(hrows, et_f, relp).reshape(NEP)
        nb = _pass2(cur.reshape(4 * N_NODES, HQ), scores, tails_p, heads_p)
        cur = _dense(cur, nb,
                     a1w[i], a1b[i].reshape(1, D),
                     a2w[i], a2b[i].reshape(1, D))
        outs.append(cur)
    return jnp.concatenate(outs, axis=1)


# speedup vs baseline: 3.2200x; 1.6320x over previous
import functools

import jax
import jax.numpy as jnp
from jax import lax
from jax.experimental import pallas as pl
from jax.experimental.pallas import tpu as pltpu
from jax.experimental.pallas import tpu_sc as plsc

N_NODES = 50000
N_REL = 17
D = 64
H = 32
NE = 800000
NEP = 819200
NW = 32
EPW1 = NEP // NW
C1 = 512
NCH1 = EPW1 // C1
EPS2 = NEP // 16
C2 = 1024
NCH2 = EPS2 // C2
ACC_ROWS = 51200
RELP_ROWS = 32

_mesh = lambda: plsc.VectorSubcoreMesh(core_axis_name="c", subcore_axis_name="s")


_RBF = 1000


def _ftab_body(x_ref, relp_ref, o_ref):
    x = x_ref[...]
    rp = relp_ref[...]
    cols = []
    for k in range(N_REL):
        rk = rp[k, :][None, :]
        cols.append(jnp.sum(x * jnp.tanh(x + rk), axis=1, keepdims=True))
    cols.append(jnp.zeros((_RBF, RELP_ROWS - N_REL), jnp.float32))
    o_ref[...] = jnp.concatenate(cols, axis=1)


def _ftab(cur, relp):
    return pl.pallas_call(
        _ftab_body,
        grid=(N_NODES // _RBF,),
        in_specs=[
            pl.BlockSpec((_RBF, D), lambda i: (i, 0)),
            pl.BlockSpec((RELP_ROWS, D), lambda i: (0, 0)),
        ],
        out_specs=pl.BlockSpec((_RBF, RELP_ROWS), lambda i: (i, 0)),
        out_shape=jax.ShapeDtypeStruct((N_NODES, RELP_ROWS), jnp.float32),
    )(cur, relp)


@functools.partial(
    pl.kernel,
    out_type=jax.ShapeDtypeStruct((NEP,), jnp.float32),
    mesh=_mesh(),
    compiler_params=pltpu.CompilerParams(needs_layout_passes=False, use_tc_tiling_on_sc=False),
    scratch_types=[
        pltpu.VMEM((C1 // 128, 128), jnp.int32),
        pltpu.VMEM((C1 // 128, 128), jnp.int32),
        pltpu.VMEM((C1,), jnp.int32),
        pltpu.VMEM((C1,), jnp.int32),
        pltpu.VMEM((C1, RELP_ROWS), jnp.float32),
        pltpu.VMEM((C1, RELP_ROWS), jnp.float32),
        pltpu.VMEM((C1,), jnp.float32),
        pltpu.VMEM((C1,), jnp.float32),
        pltpu.SemaphoreType.DMA,
        pltpu.SemaphoreType.DMA,
    ],
)
def _scorepass(ftab_hbm, heads_hbm, etype_hbm, scores_hbm,
               hidxA, hidxB, etvA, etvB, frowsA, frowsB, sbufA, sbufB,
               semA, semB):
    c = lax.axis_index("c")
    s = lax.axis_index("s")
    wid = s * 2 + c
    base_row = wid * (EPW1 // 128)
    base_edge = wid * EPW1
    NSUB = C1 // 128

    def prep(hidx, etv, frows, sem, ch):
        row0 = base_row + ch * NSUB
        off = base_edge + ch * C1
        pltpu.sync_copy(heads_hbm.at[pl.ds(row0, NSUB)], hidx)
        pltpu.sync_copy(etype_hbm.at[pl.ds(off, C1)], etv)
        for j in range(NSUB):
            pltpu.async_copy(ftab_hbm.at[hidx.at[j]],
                             frows.at[pl.ds(j * 128, 128)], sem)

    def drain(hidx, frows, sem):
        for j in range(NSUB):
            pltpu.make_async_copy(ftab_hbm.at[hidx.at[j]],
                                  frows.at[pl.ds(j * 128, 128)], sem).wait()

    def compute(etv, frows, sbuf, ch):
        off = base_edge + ch * C1

        def group_body(g, carry2):
            e0 = g * 16
            eloc = e0 + lax.iota(jnp.int32, 16)
            et = etv[pl.ds(e0, 16)]
            sc = plsc.load_gather(frows, [eloc, et])
            gid = off + e0 + lax.iota(jnp.int32, 16)
            sbuf[pl.ds(e0, 16)] = jnp.where(gid < NE, sc, 0.0)
            return carry2

        lax.fori_loop(0, C1 // 16, group_body, 0)
        pltpu.sync_copy(sbuf, scores_hbm.at[pl.ds(off, C1)])

    prep(hidxA, etvA, frowsA, semA, 0)
    prep(hidxB, etvB, frowsB, semB, 1)

    def pair_body(i, carry):
        chA = 2 * i
        chB = 2 * i + 1
        drain(hidxA, frowsA, semA)
        compute(etvA, frowsA, sbufA, chA)
        prep(hidxA, etvA, frowsA, semA, jnp.minimum(chA + 2, NCH1 - 1))
        drain(hidxB, frowsB, semB)
        compute(etvB, frowsB, sbufB, chB)
        prep(hidxB, etvB, frowsB, semB, jnp.minimum(chB + 2, NCH1 - 1))
        return carry

    lax.fori_loop(0, NCH1 // 2, pair_body, 0)
    drain(hidxA, frowsA, semA)
    drain(hidxB, frowsB, semB)


HQ = 16


@functools.partial(
    pl.kernel,
    out_type=jax.ShapeDtypeStruct((4, N_NODES, HQ), jnp.float32),
    mesh=_mesh(),
    compiler_params=pltpu.CompilerParams(needs_layout_passes=False, use_tc_tiling_on_sc=False),
    scratch_types=[
        pltpu.VMEM((C2 // 128, 128), jnp.int32),
        pltpu.VMEM((C2 // 128, 128), jnp.int32),
        pltpu.VMEM((C2 // 128, 128), jnp.int32),
        pltpu.VMEM((C2 // 128, 128), jnp.int32),
        pltpu.VMEM((C2,), jnp.float32),
        pltpu.VMEM((C2,), jnp.float32),
        pltpu.VMEM((C2, HQ), jnp.float32),
        pltpu.VMEM((C2, HQ), jnp.float32),
        pltpu.VMEM((C2, HQ), jnp.float32),
        pltpu.VMEM((C2, HQ), jnp.float32),
        pltpu.VMEM_SHARED((ACC_ROWS, HQ), jnp.float32),
        pltpu.SemaphoreType.DMA,
        pltpu.SemaphoreType.DMA,
        pltpu.SemaphoreType.DMA,
    ],
)
def _pass2(cur4_hbm, scores_hbm, tails_hbm, heads_hbm, out_hbm,
           tidxA, tidxB, hidxA, hidxB, svA, svB, trowsA, trowsB,
           wrowsA, wrowsB, acc, semA, semB, semS):
    c = lax.axis_index("c")
    s = lax.axis_index("s")
    base_row = s * (EPS2 // 128)
    base_edge = s * EPS2
    zero16 = jnp.zeros((16,), jnp.float32)
    NSUB = C2 // 128

    for qh in range(2):
        quarter = c * 2 + qh
        plsc.subcore_barrier()
        for r in range(128):
            wrowsA[r, pl.ds(0, 16)] = zero16
        for q in range(ACC_ROWS // 16 // 128):
            pltpu.sync_copy(wrowsA.at[pl.ds(0, 128)],
                            acc.at[pl.ds(s * (ACC_ROWS // 16) + q * 128, 128)])
        plsc.subcore_barrier()

        def prep(tidx, hidx, sv, trows, sem, ch):
            row0 = base_row + ch * NSUB
            off = base_edge + ch * C2
            pltpu.sync_copy(tails_hbm.at[pl.ds(row0, NSUB)], tidx)
            pltpu.sync_copy(heads_hbm.at[pl.ds(row0, NSUB)], hidx)
            pltpu.sync_copy(scores_hbm.at[pl.ds(off, C2)], sv)
            for j in range(NSUB):
                for k in range(8):
                    v = tidx[j, pl.ds(k * 16, 16)]
                    tidx[j, pl.ds(k * 16, 16)] = v * 4 + quarter
            for j in range(NSUB):
                pltpu.async_copy(cur4_hbm.at[tidx.at[j]],
                                 trows.at[pl.ds(j * 128, 128)], sem)

        def drain(tidx, trows, sem):
            for j in range(NSUB):
                pltpu.make_async_copy(cur4_hbm.at[tidx.at[j]],
                                      trows.at[pl.ds(j * 128, 128)],
                                      sem).wait()

        def compute(sv, trows, wrows):
            def wgroup(g, carry2):
                e0 = g * 16
                w = sv[pl.ds(e0, 16)]
                eloc = e0 + lax.iota(jnp.int32, 16)
                for d in range(HQ):
                    dsp = jnp.full((16,), d, jnp.int32)
                    tv = plsc.load_gather(trows, [eloc, dsp])
                    plsc.store_scatter(wrows, [eloc, dsp], tv * w)
                return carry2

            lax.fori_loop(0, C2 // 16, wgroup, 0)

        def scatter(wrows, hidx):
            for j in range(NSUB):
                pltpu.async_copy(wrows.at[pl.ds(j * 128, 128)],
                                 acc.at[hidx.at[j]], semS, add=True)
            for j in range(NSUB):
                pltpu.make_async_copy(wrows.at[pl.ds(j * 128, 128)],
                                      acc.at[hidx.at[j]], semS).wait()

        prep(tidxA, hidxA, svA, trowsA, semA, 0)
        prep(tidxB, hidxB, svB, trowsB, semB, 1)

        def pair_body(i, carry):
            chA = 2 * i
            chB = 2 * i + 1
            drain(tidxA, trowsA, semA)
            compute(svA, trowsA, wrowsA)
            scatter(wrowsA, hidxA)
            prep(tidxA, hidxA, svA, trowsA, semA,
                 jnp.minimum(chA + 2, NCH2 - 1))
            drain(tidxB, trowsB, semB)
            compute(svB, trowsB, wrowsB)
            scatter(wrowsB, hidxB)
            prep(tidxB, hidxB, svB, trowsB, semB,
                 jnp.minimum(chB + 2, NCH2 - 1))
            return carry

        lax.fori_loop(0, NCH2 // 2, pair_body, 0)
        drain(tidxA, trowsA, semA)
        drain(tidxB, trowsB, semB)
        plsc.subcore_barrier()

        r0 = s * (N_NODES // 16)
        for sz, o in ((1024, 0), (1024, 1024), (1024, 2048), (53, 3072)):
            pltpu.sync_copy(acc.at[pl.ds(r0 + o, sz)],
                            out_hbm.at[quarter, pl.ds(r0 + o, sz)])


def _relp_body(rt_ref, w_ref, b_ref, o_ref):
    o_ref[...] = (
        jnp.dot(rt_ref[...], w_ref[...].T, preferred_element_type=jnp.float32)
        + b_ref[...]
    )


def _relproj(rt_pad, w, b):
    return pl.pallas_call(
        _relp_body,
        out_shape=jax.ShapeDtypeStruct((RELP_ROWS, D), jnp.float32),
    )(rt_pad, w, b)


_RB = 1000


def _dense_body(x_ref, n0_ref, n1_ref, n2_ref, n3_ref,
                w1_ref, b1_ref, w2_ref, b2_ref, o_ref):
    x = x_ref[...]
    nb = jnp.concatenate(
        [n0_ref[...], n1_ref[...], n2_ref[...], n3_ref[...]], axis=1)
    y = (
        jnp.dot(x + nb, w1_ref[...].T, preferred_element_type=jnp.float32)
        + jnp.dot(x * nb, w2_ref[...].T, preferred_element_type=jnp.float32)
        + b1_ref[...] + b2_ref[...]
    )
    y = jnp.where(y >= 0, y, 0.01 * y)
    n = jnp.sqrt(jnp.sum(y * y, axis=1, keepdims=True))
    o_ref[...] = y / jnp.maximum(n, 1e-12)


def _dense(x, nbq, w1, b1, w2, b2):
    grid = N_NODES // _RB
    bs = lambda shp: pl.BlockSpec(shp, lambda i: (i, 0))
    const = lambda shp: pl.BlockSpec(shp, lambda i: (0, 0))
    return pl.pallas_call(
        _dense_body,
        grid=(grid,),
        in_specs=[
            bs((_RB, D)),
            bs((_RB, HQ)), bs((_RB, HQ)), bs((_RB, HQ)), bs((_RB, HQ)),
            const((D, D)), const((1, D)), const((D, D)), const((1, D)),
        ],
        out_specs=bs((_RB, D)),
        out_shape=jax.ShapeDtypeStruct((N_NODES, D), jnp.float32),
    )(x, nbq[0], nbq[1], nbq[2], nbq[3], w1, b1, w2, b2)


def kernel(entity_table, relation_table, rp_w0, rp_b0, rp_w1, rp_b1,
           a1w0, a1b0, a2w0, a2b0, a1w1, a1b1, a2w1, a2b1,
           edge_index, edge_type):
    heads = edge_index[0].astype(jnp.int32)
    tails = edge_index[1].astype(jnp.int32)
    et = edge_type.astype(jnp.int32)
    padz = jnp.zeros((NEP - NE,), jnp.int32)
    heads_p = jnp.concatenate([heads, padz]).reshape(NEP // 128, 128)
    tails_p = jnp.concatenate([tails, padz]).reshape(NEP // 128, 128)
    et_p = jnp.concatenate([et, padz])
    rt_pad = jnp.concatenate(
        [relation_table,
         jnp.zeros((RELP_ROWS - N_REL, D), jnp.float32)], axis=0)

    rp_w = [rp_w0, rp_w1]
    rp_b = [rp_b0, rp_b1]
    a1w = [a1w0, a1w1]
    a1b = [a1b0, a1b1]
    a2w = [a2w0, a2w1]
    a2b = [a2b0, a2b1]

    cur = entity_table
    outs = [cur]
    for i in range(2):
        relp = _relproj(rt_pad, rp_w[i], rp_b[i].reshape(1, D))
        ftab = _ftab(cur, relp)
        scores = _scorepass(ftab, heads_p, et_p)
        nb = _pass2(cur.reshape(4 * N_NODES, HQ), scores, tails_p, heads_p)
        cur = _dense(cur, nb,
                     a1w[i], a1b[i].reshape(1, D),
                     a2w[i], a2b[i].reshape(1, D))
        outs.append(cur)
    return jnp.concatenate(outs, axis=1)


# speedup vs baseline: 3.2586x; 1.0120x over previous
import functools

import jax
import jax.numpy as jnp
from jax import lax
from jax.experimental import pallas as pl
from jax.experimental.pallas import tpu as pltpu
from jax.experimental.pallas import tpu_sc as plsc

N_NODES = 50000
N_REL = 17
D = 64
H = 32
NE = 800000
NEP = 819200
NW = 32
EPW1 = NEP // NW
C1 = 512
NCH1 = EPW1 // C1
EPS2 = NEP // 16
C2 = 1024
NCH2 = EPS2 // C2
ACC_ROWS = 51200
RELP_ROWS = 32

_mesh = lambda: plsc.VectorSubcoreMesh(core_axis_name="c", subcore_axis_name="s")


_RBF = 1000


def _ftab_body(x_ref, relp_ref, o_ref):
    x = x_ref[...]
    rp = relp_ref[...]
    cols = []
    for k in range(N_REL):
        rk = rp[k, :][None, :]
        cols.append(jnp.sum(x * jnp.tanh(x + rk), axis=1, keepdims=True))
    cols.append(jnp.zeros((_RBF, RELP_ROWS - N_REL), jnp.float32))
    o_ref[...] = jnp.concatenate(cols, axis=1)


def _ftab(cur, relp):
    return pl.pallas_call(
        _ftab_body,
        grid=(N_NODES // _RBF,),
        in_specs=[
            pl.BlockSpec((_RBF, D), lambda i: (i, 0)),
            pl.BlockSpec((RELP_ROWS, D), lambda i: (0, 0)),
        ],
        out_specs=pl.BlockSpec((_RBF, RELP_ROWS), lambda i: (i, 0)),
        out_shape=jax.ShapeDtypeStruct((N_NODES, RELP_ROWS), jnp.float32),
    )(cur, relp)


@functools.partial(
    pl.kernel,
    out_type=jax.ShapeDtypeStruct((NEP,), jnp.float32),
    mesh=_mesh(),
    compiler_params=pltpu.CompilerParams(needs_layout_passes=False, use_tc_tiling_on_sc=False),
    scratch_types=[
        pltpu.VMEM((C1 // 128, 128), jnp.int32),
        pltpu.VMEM((C1 // 128, 128), jnp.int32),
        pltpu.VMEM((C1,), jnp.int32),
        pltpu.VMEM((C1,), jnp.int32),
        pltpu.VMEM((C1, RELP_ROWS), jnp.float32),
        pltpu.VMEM((C1, RELP_ROWS), jnp.float32),
        pltpu.VMEM((C1,), jnp.float32),
        pltpu.VMEM((C1,), jnp.float32),
        pltpu.SemaphoreType.DMA,
        pltpu.SemaphoreType.DMA,
    ],
)
def _scorepass(ftab_hbm, heads_hbm, etype_hbm, scores_hbm,
               hidxA, hidxB, etvA, etvB, frowsA, frowsB, sbufA, sbufB,
               semA, semB):
    c = lax.axis_index("c")
    s = lax.axis_index("s")
    wid = s * 2 + c
    base_row = wid * (EPW1 // 128)
    base_edge = wid * EPW1
    NSUB = C1 // 128

    def prep(hidx, etv, frows, sem, ch):
        row0 = base_row + ch * NSUB
        off = base_edge + ch * C1
        pltpu.sync_copy(heads_hbm.at[pl.ds(row0, NSUB)], hidx)
        pltpu.sync_copy(etype_hbm.at[pl.ds(off, C1)], etv)
        for j in range(NSUB):
            pltpu.async_copy(ftab_hbm.at[hidx.at[j]],
                             frows.at[pl.ds(j * 128, 128)], sem)

    def drain(hidx, frows, sem):
        for j in range(NSUB):
            pltpu.make_async_copy(ftab_hbm.at[hidx.at[j]],
                                  frows.at[pl.ds(j * 128, 128)], sem).wait()

    def compute(etv, frows, sbuf, ch):
        off = base_edge + ch * C1

        def group_body(g, carry2):
            e0 = g * 16
            eloc = e0 + lax.iota(jnp.int32, 16)
            et = etv[pl.ds(e0, 16)]
            sc = plsc.load_gather(frows, [eloc, et])
            gid = off + e0 + lax.iota(jnp.int32, 16)
            sbuf[pl.ds(e0, 16)] = jnp.where(gid < NE, sc, 0.0)
            return carry2

        lax.fori_loop(0, C1 // 16, group_body, 0)
        pltpu.sync_copy(sbuf, scores_hbm.at[pl.ds(off, C1)])

    prep(hidxA, etvA, frowsA, semA, 0)
    prep(hidxB, etvB, frowsB, semB, 1)

    def pair_body(i, carry):
        chA = 2 * i
        chB = 2 * i + 1
        drain(hidxA, frowsA, semA)
        compute(etvA, frowsA, sbufA, chA)
        prep(hidxA, etvA, frowsA, semA, jnp.minimum(chA + 2, NCH1 - 1))
        drain(hidxB, frowsB, semB)
        compute(etvB, frowsB, sbufB, chB)
        prep(hidxB, etvB, frowsB, semB, jnp.minimum(chB + 2, NCH1 - 1))
        return carry

    lax.fori_loop(0, NCH1 // 2, pair_body, 0)
    drain(hidxA, frowsA, semA)
    drain(hidxB, frowsB, semB)


HQ = 16


@functools.partial(
    pl.kernel,
    out_type=jax.ShapeDtypeStruct((4, N_NODES, HQ), jnp.float32),
    mesh=_mesh(),
    compiler_params=pltpu.CompilerParams(needs_layout_passes=False, use_tc_tiling_on_sc=False),
    scratch_types=[
        pltpu.VMEM((C2 // 128, 128), jnp.int32),
        pltpu.VMEM((C2 // 128, 128), jnp.int32),
        pltpu.VMEM((C2 // 128, 128), jnp.int32),
        pltpu.VMEM((C2 // 128, 128), jnp.int32),
        pltpu.VMEM((C2,), jnp.float32),
        pltpu.VMEM((C2,), jnp.float32),
        pltpu.VMEM((C2, HQ), jnp.float32),
        pltpu.VMEM((C2, HQ), jnp.float32),
        pltpu.VMEM((C2, HQ), jnp.float32),
        pltpu.VMEM((C2, HQ), jnp.float32),
        pltpu.VMEM_SHARED((ACC_ROWS, HQ), jnp.float32),
        pltpu.SemaphoreType.DMA,
        pltpu.SemaphoreType.DMA,
        pltpu.SemaphoreType.DMA,
        pltpu.SemaphoreType.DMA,
    ],
)
def _pass2(cur4_hbm, scores_hbm, tails_hbm, heads_hbm, out_hbm,
           tidxA, tidxB, hidxA, hidxB, svA, svB, trowsA, trowsB,
           wrowsA, wrowsB, acc, semA, semB, semSA, semSB):
    c = lax.axis_index("c")
    s = lax.axis_index("s")
    base_row = s * (EPS2 // 128)
    base_edge = s * EPS2
    zero16 = jnp.zeros((16,), jnp.float32)
    NSUB = C2 // 128

    for qh in range(2):
        quarter = c * 2 + qh
        plsc.subcore_barrier()
        for r in range(128):
            wrowsA[r, pl.ds(0, 16)] = zero16
        for q in range(ACC_ROWS // 16 // 128):
            pltpu.sync_copy(wrowsA.at[pl.ds(0, 128)],
                            acc.at[pl.ds(s * (ACC_ROWS // 16) + q * 128, 128)])
        plsc.subcore_barrier()

        def prep_gather(tidx, trows, sem, ch):
            row0 = base_row + ch * NSUB
            pltpu.sync_copy(tails_hbm.at[pl.ds(row0, NSUB)], tidx)
            for j in range(NSUB):
                for k in range(8):
                    v = tidx[j, pl.ds(k * 16, 16)]
                    tidx[j, pl.ds(k * 16, 16)] = v * 4 + quarter
            for j in range(NSUB):
                pltpu.async_copy(cur4_hbm.at[tidx.at[j]],
                                 trows.at[pl.ds(j * 128, 128)], sem)

        def drain_gather(tidx, trows, sem):
            for j in range(NSUB):
                pltpu.make_async_copy(cur4_hbm.at[tidx.at[j]],
                                      trows.at[pl.ds(j * 128, 128)],
                                      sem).wait()

        def compute(sv, trows, wrows):
            def wgroup(g, carry2):
                e0 = g * 16
                w = sv[pl.ds(e0, 16)]
                eloc = e0 + lax.iota(jnp.int32, 16)
                for d in range(HQ):
                    dsp = jnp.full((16,), d, jnp.int32)
                    tv = plsc.load_gather(trows, [eloc, dsp])
                    plsc.store_scatter(wrows, [eloc, dsp], tv * w)
                return carry2

            lax.fori_loop(0, C2 // 16, wgroup, 0)

        def fire_scatter(wrows, hidx, semS):
            for j in range(NSUB):
                pltpu.async_copy(wrows.at[pl.ds(j * 128, 128)],
                                 acc.at[hidx.at[j]], semS, add=True)

        def drain_scatter(wrows, hidx, semS):
            for j in range(NSUB):
                pltpu.make_async_copy(wrows.at[pl.ds(j * 128, 128)],
                                      acc.at[hidx.at[j]], semS).wait()

        def phase(i, ch, tidx, hidx, sv, trows, wrows, sem, semS):
            @pl.when(i > 0)
            def _():
                drain_scatter(wrows, hidx, semS)

            row0 = base_row + ch * NSUB
            off = base_edge + ch * C2
            pltpu.sync_copy(heads_hbm.at[pl.ds(row0, NSUB)], hidx)
            pltpu.sync_copy(scores_hbm.at[pl.ds(off, C2)], sv)
            drain_gather(tidx, trows, sem)
            compute(sv, trows, wrows)
            fire_scatter(wrows, hidx, semS)
            prep_gather(tidx, trows, sem, jnp.minimum(ch + 2, NCH2 - 1))

        prep_gather(tidxA, trowsA, semA, 0)
        prep_gather(tidxB, trowsB, semB, 1)

        def pair_body(i, carry):
            phase(i, 2 * i, tidxA, hidxA, svA, trowsA, wrowsA, semA, semSA)
            phase(i, 2 * i + 1, tidxB, hidxB, svB, trowsB, wrowsB, semB, semSB)
            return carry

        lax.fori_loop(0, NCH2 // 2, pair_body, 0)
        drain_scatter(wrowsA, hidxA, semSA)
        drain_scatter(wrowsB, hidxB, semSB)
        drain_gather(tidxA, trowsA, semA)
        drain_gather(tidxB, trowsB, semB)
        plsc.subcore_barrier()

        r0 = s * (N_NODES // 16)
        for sz, o in ((1024, 0), (1024, 1024), (1024, 2048), (53, 3072)):
            pltpu.sync_copy(acc.at[pl.ds(r0 + o, sz)],
                            out_hbm.at[quarter, pl.ds(r0 + o, sz)])


def _relp_body(rt_ref, w_ref, b_ref, o_ref):
    o_ref[...] = (
        jnp.dot(rt_ref[...], w_ref[...].T, preferred_element_type=jnp.float32)
        + b_ref[...]
    )


def _relproj(rt_pad, w, b):
    return pl.pallas_call(
        _relp_body,
        out_shape=jax.ShapeDtypeStruct((RELP_ROWS, D), jnp.float32),
    )(rt_pad, w, b)


_RB = 1000


def _dense_body(x_ref, n0_ref, n1_ref, n2_ref, n3_ref,
                w1_ref, b1_ref, w2_ref, b2_ref, o_ref):
    x = x_ref[...]
    nb = jnp.concatenate(
        [n0_ref[...], n1_ref[...], n2_ref[...], n3_ref[...]], axis=1)
    y = (
        jnp.dot(x + nb, w1_ref[...].T, preferred_element_type=jnp.float32)
        + jnp.dot(x * nb, w2_ref[...].T, preferred_element_type=jnp.float32)
        + b1_ref[...] + b2_ref[...]
    )
    y = jnp.where(y >= 0, y, 0.01 * y)
    n = jnp.sqrt(jnp.sum(y * y, axis=1, keepdims=True))
    o_ref[...] = y / jnp.maximum(n, 1e-12)


def _dense(x, nbq, w1, b1, w2, b2):
    grid = N_NODES // _RB
    bs = lambda shp: pl.BlockSpec(shp, lambda i: (i, 0))
    const = lambda shp: pl.BlockSpec(shp, lambda i: (0, 0))
    return pl.pallas_call(
        _dense_body,
        grid=(grid,),
        in_specs=[
            bs((_RB, D)),
            bs((_RB, HQ)), bs((_RB, HQ)), bs((_RB, HQ)), bs((_RB, HQ)),
            const((D, D)), const((1, D)), const((D, D)), const((1, D)),
        ],
        out_specs=bs((_RB, D)),
        out_shape=jax.ShapeDtypeStruct((N_NODES, D), jnp.float32),
    )(x, nbq[0], nbq[1], nbq[2], nbq[3], w1, b1, w2, b2)


def kernel(entity_table, relation_table, rp_w0, rp_b0, rp_w1, rp_b1,
           a1w0, a1b0, a2w0, a2b0, a1w1, a1b1, a2w1, a2b1,
           edge_index, edge_type):
    heads = edge_index[0].astype(jnp.int32)
    tails = edge_index[1].astype(jnp.int32)
    et = edge_type.astype(jnp.int32)
    padz = jnp.zeros((NEP - NE,), jnp.int32)
    heads_p = jnp.concatenate([heads, padz]).reshape(NEP // 128, 128)
    tails_p = jnp.concatenate([tails, padz]).reshape(NEP // 128, 128)
    et_p = jnp.concatenate([et, padz])
    rt_pad = jnp.concatenate(
        [relation_table,
         jnp.zeros((RELP_ROWS - N_REL, D), jnp.float32)], axis=0)

    rp_w = [rp_w0, rp_w1]
    rp_b = [rp_b0, rp_b1]
    a1w = [a1w0, a1w1]
    a1b = [a1b0, a1b1]
    a2w = [a2w0, a2w1]
    a2b = [a2b0, a2b1]

    cur = entity_table
    outs = [cur]
    for i in range(2):
        relp = _relproj(rt_pad, rp_w[i], rp_b[i].reshape(1, D))
        ftab = _ftab(cur, relp)
        scores = _scorepass(ftab, heads_p, et_p)
        nb = _pass2(cur.reshape(4 * N_NODES, HQ), scores, tails_p, heads_p)
        cur = _dense(cur, nb,
                     a1w[i], a1b[i].reshape(1, D),
                     a2w[i], a2b[i].reshape(1, D))
        outs.append(cur)
    return jnp.concatenate(outs, axis=1)


# speedup vs baseline: 4.0331x; 1.2377x over previous
import functools

import jax
import jax.numpy as jnp
from jax import lax
from jax.experimental import pallas as pl
from jax.experimental.pallas import tpu as pltpu
from jax.experimental.pallas import tpu_sc as plsc

N_NODES = 50000
N_REL = 17
D = 64
H = 32
NE = 800000
NEP = 819200
NW = 32
EPW1 = NEP // NW
C1 = 512
NCH1 = EPW1 // C1
EPS2 = NEP // 16
C2 = 512
NCH2 = EPS2 // C2
ACC_ROWS = 51200
RELP_ROWS = 32

_mesh = lambda: plsc.VectorSubcoreMesh(core_axis_name="c", subcore_axis_name="s")


_RBF = 1000


def _ftab_body(x_ref, relp_ref, o_ref):
    x = x_ref[...]
    rp = relp_ref[...]
    cols = []
    for k in range(N_REL):
        rk = rp[k, :][None, :]
        cols.append(jnp.sum(x * jnp.tanh(x + rk), axis=1, keepdims=True))
    cols.append(jnp.zeros((_RBF, RELP_ROWS - N_REL), jnp.float32))
    o_ref[...] = jnp.concatenate(cols, axis=1)


def _ftab(cur, relp):
    return pl.pallas_call(
        _ftab_body,
        grid=(N_NODES // _RBF,),
        in_specs=[
            pl.BlockSpec((_RBF, D), lambda i: (i, 0)),
            pl.BlockSpec((RELP_ROWS, D), lambda i: (0, 0)),
        ],
        out_specs=pl.BlockSpec((_RBF, RELP_ROWS), lambda i: (i, 0)),
        out_shape=jax.ShapeDtypeStruct((N_NODES, RELP_ROWS), jnp.float32),
    )(cur, relp)


@functools.partial(
    pl.kernel,
    out_type=jax.ShapeDtypeStruct((NEP // 128, 128), jnp.float32),
    mesh=_mesh(),
    compiler_params=pltpu.CompilerParams(needs_layout_passes=False, use_tc_tiling_on_sc=False),
    scratch_types=[
        pltpu.VMEM((C1 // 128, 128), jnp.int32),
        pltpu.VMEM((C1 // 128, 128), jnp.int32),
        pltpu.VMEM((C1,), jnp.int32),
        pltpu.VMEM((C1,), jnp.int32),
        pltpu.VMEM((C1, RELP_ROWS), jnp.float32),
        pltpu.VMEM((C1, RELP_ROWS), jnp.float32),
        pltpu.VMEM((C1 // 128, 128), jnp.float32),
        pltpu.VMEM((C1 // 128, 128), jnp.float32),
        pltpu.SemaphoreType.DMA,
        pltpu.SemaphoreType.DMA,
    ],
)
def _scorepass(ftab_hbm, heads_hbm, etype_hbm, scores_hbm,
               hidxA, hidxB, etvA, etvB, frowsA, frowsB, sbufA, sbufB,
               semA, semB):
    c = lax.axis_index("c")
    s = lax.axis_index("s")
    wid = s * 2 + c
    base_row = wid * (EPW1 // 128)
    base_edge = wid * EPW1
    NSUB = C1 // 128

    def prep(hidx, etv, frows, sem, ch):
        row0 = base_row + ch * NSUB
        off = base_edge + ch * C1
        pltpu.sync_copy(heads_hbm.at[pl.ds(row0, NSUB)], hidx)
        pltpu.sync_copy(etype_hbm.at[pl.ds(off, C1)], etv)
        for j in range(NSUB):
            pltpu.async_copy(ftab_hbm.at[hidx.at[j]],
                             frows.at[pl.ds(j * 128, 128)], sem)

    def drain(hidx, frows, sem):
        for j in range(NSUB):
            pltpu.make_async_copy(ftab_hbm.at[hidx.at[j]],
                                  frows.at[pl.ds(j * 128, 128)], sem).wait()

    def compute(etv, frows, sbuf, ch):
        off = base_edge + ch * C1
        row0 = base_row + ch * NSUB

        def group_body(g, carry2):
            e0 = g * 16
            eloc = e0 + lax.iota(jnp.int32, 16)
            et = etv[pl.ds(e0, 16)]
            sc = plsc.load_gather(frows, [eloc, et])
            gid = off + e0 + lax.iota(jnp.int32, 16)
            sc = jnp.where(gid < NE, sc, 0.0)
            plsc.store_scatter(sbuf, [eloc // 128, eloc % 128], sc)
            return carry2

        lax.fori_loop(0, C1 // 16, group_body, 0)
        pltpu.sync_copy(sbuf, scores_hbm.at[pl.ds(row0, NSUB)])

    prep(hidxA, etvA, frowsA, semA, 0)
    prep(hidxB, etvB, frowsB, semB, 1)

    def pair_body(i, carry):
        chA = 2 * i
        chB = 2 * i + 1
        drain(hidxA, frowsA, semA)
        compute(etvA, frowsA, sbufA, chA)
        prep(hidxA, etvA, frowsA, semA, jnp.minimum(chA + 2, NCH1 - 1))
        drain(hidxB, frowsB, semB)
        compute(etvB, frowsB, sbufB, chB)
        prep(hidxB, etvB, frowsB, semB, jnp.minimum(chB + 2, NCH1 - 1))
        return carry

    lax.fori_loop(0, NCH1 // 2, pair_body, 0)
    drain(hidxA, frowsA, semA)
    drain(hidxB, frowsB, semB)


HH = 32


@functools.partial(
    pl.kernel,
    out_type=jax.ShapeDtypeStruct((2, N_NODES, HH), jnp.bfloat16),
    mesh=_mesh(),
    compiler_params=pltpu.CompilerParams(needs_layout_passes=False, use_tc_tiling_on_sc=False),
    scratch_types=[
        pltpu.VMEM((C2 // 128, 128), jnp.int32),
        pltpu.VMEM((C2 // 128, 128), jnp.int32),
        pltpu.VMEM((C2 // 128, 128), jnp.int32),
        pltpu.VMEM((C2 // 128, 128), jnp.int32),
        pltpu.VMEM((C2 // 128, 128), jnp.float32),
        pltpu.VMEM((C2 // 128, 128), jnp.float32),
        pltpu.VMEM((C2, HH), jnp.float32),
        pltpu.VMEM((C2, HH), jnp.float32),
        pltpu.VMEM((C2, HH), jnp.bfloat16),
        pltpu.VMEM((C2, HH), jnp.bfloat16),
        pltpu.VMEM_SHARED((ACC_ROWS, HH), jnp.bfloat16),
        pltpu.SemaphoreType.DMA,
        pltpu.SemaphoreType.DMA,
        pltpu.SemaphoreType.DMA,
        pltpu.SemaphoreType.DMA,
    ],
)
def _pass2(cur2_hbm, scores_hbm, tails_hbm, heads_hbm, out_hbm,
           tidxA, tidxB, hidxA, hidxB, svA, svB, trowsA, trowsB,
           wrowsA, wrowsB, acc, semA, semB, semSA, semSB):
    c = lax.axis_index("c")
    s = lax.axis_index("s")
    base_row = s * (EPS2 // 128)
    NSUB = C2 // 128
    zero32 = jnp.zeros((HH,), jnp.bfloat16)

    for r in range(128):
        wrowsA[r, pl.ds(0, HH)] = zero32
    for q in range(ACC_ROWS // 16 // 128):
        pltpu.sync_copy(wrowsA.at[pl.ds(0, 128)],
                        acc.at[pl.ds(s * (ACC_ROWS // 16) + q * 128, 128)])
    plsc.subcore_barrier()

    def prep_gather(tidx, trows, sem, ch):
        row0 = base_row + ch * NSUB
        pltpu.sync_copy(tails_hbm.at[pl.ds(row0, NSUB)], tidx)
        for j in range(NSUB):
            for k in range(8):
                v = tidx[j, pl.ds(k * 16, 16)]
                tidx[j, pl.ds(k * 16, 16)] = v + v + c
        for j in range(NSUB):
            pltpu.async_copy(cur2_hbm.at[tidx.at[j]],
                             trows.at[pl.ds(j * 128, 128)], sem)

    def drain_gather(tidx, trows, sem):
        for j in range(NSUB):
            pltpu.make_async_copy(cur2_hbm.at[tidx.at[j]],
                                  trows.at[pl.ds(j * 128, 128)],
                                  sem).wait()

    idx_even = lax.iota(jnp.int32, 16) * 2
    idx_odd = idx_even + 1

    def compute(sv, trows, wrows):
        def wedge(e, carry2):
            ev = jnp.zeros((16,), jnp.int32) + e
            w = plsc.load_gather(sv, [ev // 128, ev % 128])
            a = plsc.load_gather(trows, [ev, idx_even])
            b = plsc.load_gather(trows, [ev, idx_odd])
            wrows[e, pl.ds(0, HH)] = plsc.pack(
                a * w, b * w, format=plsc.PackFormat.INTERLEAVED)
            return carry2

        lax.fori_loop(0, C2, wedge, 0)

    def fire_scatter(wrows, hidx, semS):
        for j in range(NSUB):
            pltpu.async_copy(wrows.at[pl.ds(j * 128, 128)],
                             acc.at[hidx.at[j]], semS, add=True)

    def drain_scatter(wrows, hidx, semS):
        for j in range(NSUB):
            pltpu.make_async_copy(wrows.at[pl.ds(j * 128, 128)],
                                  acc.at[hidx.at[j]], semS).wait()

    def phase(i, ch, tidx, hidx, sv, trows, wrows, sem, semS):
        @pl.when(i > 0)
        def _():
            drain_scatter(wrows, hidx, semS)

        row0 = base_row + ch * NSUB
        pltpu.sync_copy(heads_hbm.at[pl.ds(row0, NSUB)], hidx)
        pltpu.sync_copy(scores_hbm.at[pl.ds(row0, NSUB)], sv)
        drain_gather(tidx, trows, sem)
        compute(sv, trows, wrows)
        fire_scatter(wrows, hidx, semS)
        prep_gather(tidx, trows, sem, jnp.minimum(ch + 2, NCH2 - 1))

    prep_gather(tidxA, trowsA, semA, 0)
    prep_gather(tidxB, trowsB, semB, 1)

    def pair_body(i, carry):
        phase(i, 2 * i, tidxA, hidxA, svA, trowsA, wrowsA, semA, semSA)
        phase(i, 2 * i + 1, tidxB, hidxB, svB, trowsB, wrowsB, semB, semSB)
        return carry

    lax.fori_loop(0, NCH2 // 2, pair_body, 0)
    drain_scatter(wrowsA, hidxA, semSA)
    drain_scatter(wrowsB, hidxB, semSB)
    drain_gather(tidxA, trowsA, semA)
    drain_gather(tidxB, trowsB, semB)
    plsc.subcore_barrier()

    r0 = s * (N_NODES // 16)
    for sz, o in ((1024, 0), (1024, 1024), (1024, 2048), (53, 3072)):
        pltpu.sync_copy(acc.at[pl.ds(r0 + o, sz)],
                        out_hbm.at[c, pl.ds(r0 + o, sz)])


def _relp_body(rt_ref, w_ref, b_ref, o_ref):
    o_ref[...] = (
        jnp.dot(rt_ref[...], w_ref[...].T, preferred_element_type=jnp.float32)
        + b_ref[...]
    )


def _relproj(rt_pad, w, b):
    return pl.pallas_call(
        _relp_body,
        out_shape=jax.ShapeDtypeStruct((RELP_ROWS, D), jnp.float32),
    )(rt_pad, w, b)


_RB = 1000


def _dense_body(x_ref, n0_ref, n1_ref,
                w1_ref, b1_ref, w2_ref, b2_ref, o_ref):
    x = x_ref[...]
    nb = jnp.concatenate([n0_ref[...], n1_ref[...]], axis=1)
    y = (
        jnp.dot(x + nb, w1_ref[...].T, preferred_element_type=jnp.float32)
        + jnp.dot(x * nb, w2_ref[...].T, preferred_element_type=jnp.float32)
        + b1_ref[...] + b2_ref[...]
    )
    y = jnp.where(y >= 0, y, 0.01 * y)
    n = jnp.sqrt(jnp.sum(y * y, axis=1, keepdims=True))
    o_ref[...] = y / jnp.maximum(n, 1e-12)


def _dense(x, nbq, w1, b1, w2, b2):
    grid = N_NODES // _RB
    bs = lambda shp: pl.BlockSpec(shp, lambda i: (i, 0))
    const = lambda shp: pl.BlockSpec(shp, lambda i: (0, 0))
    return pl.pallas_call(
        _dense_body,
        grid=(grid,),
        in_specs=[
            bs((_RB, D)), bs((_RB, HH)), bs((_RB, HH)),
            const((D, D)), const((1, D)), const((D, D)), const((1, D)),
        ],
        out_specs=bs((_RB, D)),
        out_shape=jax.ShapeDtypeStruct((N_NODES, D), jnp.float32),
    )(x, nbq[0], nbq[1], w1, b1, w2, b2)


def kernel(entity_table, relation_table, rp_w0, rp_b0, rp_w1, rp_b1,
           a1w0, a1b0, a2w0, a2b0, a1w1, a1b1, a2w1, a2b1,
           edge_index, edge_type):
    heads = edge_index[0].astype(jnp.int32)
    tails = edge_index[1].astype(jnp.int32)
    et = edge_type.astype(jnp.int32)
    padz = jnp.zeros((NEP - NE,), jnp.int32)
    heads_p = jnp.concatenate([heads, padz]).reshape(NEP // 128, 128)
    tails_p = jnp.concatenate([tails, padz]).reshape(NEP // 128, 128)
    et_p = jnp.concatenate([et, padz])
    rt_pad = jnp.concatenate(
        [relation_table,
         jnp.zeros((RELP_ROWS - N_REL, D), jnp.float32)], axis=0)

    rp_w = [rp_w0, rp_w1]
    rp_b = [rp_b0, rp_b1]
    a1w = [a1w0, a1w1]
    a1b = [a1b0, a1b1]
    a2w = [a2w0, a2w1]
    a2b = [a2b0, a2b1]

    cur = entity_table
    outs = [cur]
    for i in range(2):
        relp = _relproj(rt_pad, rp_w[i], rp_b[i].reshape(1, D))
        ftab = _ftab(cur, relp)
        scores = _scorepass(ftab, heads_p, et_p)
        nb = _pass2(cur.reshape(2 * N_NODES, HH), scores, tails_p,
                    heads_p).astype(jnp.float32)
        cur = _dense(cur, nb,
                     a1w[i], a1b[i].reshape(1, D),
                     a2w[i], a2b[i].reshape(1, D))
        outs.append(cur)
    return jnp.concatenate(outs, axis=1)
